# baseline (speedup 1.0000x reference)
import functools

import jax
import jax.numpy as jnp
from jax import lax
from jax.experimental import pallas as pl
from jax.experimental.pallas import tpu as pltpu

BATCH = 16384
HIDDEN = 64
GAMMA = 12.0

BLK = 512
NBLK = BATCH // BLK


def _body(idx_h, idx_r, idx_t, ent_hbm, rel_hbm, out_ref,
          buf_h, buf_r, buf_t, sems):
    k = pl.program_id(0)

    def issue_block(blk, par):
        base = blk * BLK

        def enqueue(r, carry):
            ih = idx_h[base + r]
            ir = idx_r[base + r]
            it = idx_t[base + r]
            pltpu.async_copy(ent_hbm.at[ih], buf_h.at[par, r], sems.at[par])
            pltpu.async_copy(rel_hbm.at[ir], buf_r.at[par, r], sems.at[par])
            pltpu.async_copy(ent_hbm.at[it], buf_t.at[par, r], sems.at[par])
            return carry

        lax.fori_loop(0, BLK, enqueue, 0, unroll=8)

    par = lax.rem(k, 2)
    nxt = lax.rem(k + 1, 2)

    @pl.when(k == 0)
    def _():
        issue_block(0, 0)

    @pl.when(k + 1 < NBLK)
    def _():
        issue_block(k + 1, nxt)

    for buf in (buf_h, buf_r, buf_t):
        pltpu.make_async_copy(
            ent_hbm.at[pl.ds(0, BLK)], buf.at[par], sems.at[par]).wait()

    h = buf_h[par]
    r = buf_r[par]
    t = buf_t[par]
    d = jnp.abs(h + r - t)
    out_ref[...] = GAMMA - jnp.sum(d, axis=1, keepdims=True)


@jax.jit
def _score(heads, rels, tails, entity_embedding, relation_embedding):
    grid_spec = pltpu.PrefetchScalarGridSpec(
        num_scalar_prefetch=3,
        grid=(NBLK,),
        in_specs=[
            pl.BlockSpec(memory_space=pl.ANY),
            pl.BlockSpec(memory_space=pl.ANY),
        ],
        out_specs=pl.BlockSpec((BLK, 1), lambda k, *p: (k, 0)),
        scratch_shapes=[
            pltpu.VMEM((2, BLK, HIDDEN), jnp.float32),
            pltpu.VMEM((2, BLK, HIDDEN), jnp.float32),
            pltpu.VMEM((2, BLK, HIDDEN), jnp.float32),
            pltpu.SemaphoreType.DMA((2,)),
        ],
    )
    fn = pl.pallas_call(
        _body,
        grid_spec=grid_spec,
        out_shape=jax.ShapeDtypeStruct((BATCH, 1), jnp.float32),
        compiler_params=pltpu.CompilerParams(
            dimension_semantics=("arbitrary",)),
    )
    return fn(heads, rels, tails, entity_embedding, relation_embedding)


def kernel(sample, entity_embedding, relation_embedding):
    sample = sample.astype(jnp.int32)
    heads = sample[:, 0]
    rels = sample[:, 1]
    tails = sample[:, 2]
    return _score(heads, rels, tails, entity_embedding, relation_embedding)


# speedup vs baseline: 1.0757x; 1.0757x over previous
import functools

import jax
import jax.numpy as jnp
from jax import lax
from jax.experimental import pallas as pl
from jax.experimental.pallas import tpu as pltpu

BATCH = 16384
HIDDEN = 64
GAMMA = 12.0

BLK = 512
NBLK = BATCH // BLK


def _body(idx_h, idx_r, idx_t, ent_hbm, rel_hbm, out_ref,
          buf_h, buf_r, buf_t, sems):
    k = pl.program_id(0)

    def issue_block(blk, par):
        base = blk * BLK

        def enqueue(r, carry):
            ih = idx_h[base + r]
            ir = idx_r[base + r]
            it = idx_t[base + r]
            pltpu.async_copy(ent_hbm.at[ih], buf_h.at[par, r], sems.at[par])
            pltpu.async_copy(rel_hbm.at[ir], buf_r.at[par, r], sems.at[par],
                             priority=1)
            pltpu.async_copy(ent_hbm.at[it], buf_t.at[par, r], sems.at[par])
            return carry

        lax.fori_loop(0, BLK, enqueue, 0, unroll=8)

    par = lax.rem(k, 2)
    nxt = lax.rem(k + 1, 2)

    @pl.when(k == 0)
    def _():
        issue_block(0, 0)

    @pl.when(k + 1 < NBLK)
    def _():
        issue_block(k + 1, nxt)

    for buf in (buf_h, buf_r, buf_t):
        pltpu.make_async_copy(
            ent_hbm.at[pl.ds(0, BLK)], buf.at[par], sems.at[par]).wait()

    h = buf_h[par]
    r = buf_r[par]
    t = buf_t[par]
    d = jnp.abs(h + r - t)
    out_ref[...] = GAMMA - jnp.sum(d, axis=1, keepdims=True)


@jax.jit
def _score(heads, rels, tails, entity_embedding, relation_embedding):
    grid_spec = pltpu.PrefetchScalarGridSpec(
        num_scalar_prefetch=3,
        grid=(NBLK,),
        in_specs=[
            pl.BlockSpec(memory_space=pl.ANY),
            pl.BlockSpec(memory_space=pl.ANY),
        ],
        out_specs=pl.BlockSpec((BLK, 1), lambda k, *p: (k, 0)),
        scratch_shapes=[
            pltpu.VMEM((2, BLK, HIDDEN), jnp.float32),
            pltpu.VMEM((2, BLK, HIDDEN), jnp.float32),
            pltpu.VMEM((2, BLK, HIDDEN), jnp.float32),
            pltpu.SemaphoreType.DMA((2,)),
        ],
    )
    fn = pl.pallas_call(
        _body,
        grid_spec=grid_spec,
        out_shape=jax.ShapeDtypeStruct((BATCH, 1), jnp.float32),
        compiler_params=pltpu.CompilerParams(
            dimension_semantics=("arbitrary",)),
    )
    return fn(heads, rels, tails, entity_embedding, relation_embedding)


def kernel(sample, entity_embedding, relation_embedding):
    sample = sample.astype(jnp.int32)
    heads = sample[:, 0]
    rels = sample[:, 1]
    tails = sample[:, 2]
    return _score(heads, rels, tails, entity_embedding, relation_embedding)


# speedup vs baseline: 1.0974x; 1.0202x over previous
import functools

import jax
import jax.numpy as jnp
from jax import lax
from jax.experimental import pallas as pl
from jax.experimental.pallas import tpu as pltpu

BATCH = 16384
HIDDEN = 64
GAMMA = 12.0

BLK = 512
NBLK = BATCH // BLK


def _body(idx_h, idx_r, idx_t, ent_hbm, rel_hbm, out_ref,
          buf_h, buf_r, buf_t, sems):
    k = pl.program_id(0)

    def issue_block(blk, par):
        base = blk * BLK

        def enqueue(q, carry):
            for half, (ph, pr, pt) in enumerate(((0, 1, 0), (1, 0, 1))):
                r = 2 * q + half
                ih = idx_h[base + r]
                ir = idx_r[base + r]
                it = idx_t[base + r]
                pltpu.async_copy(ent_hbm.at[ih], buf_h.at[par, r],
                                 sems.at[par], priority=ph)
                pltpu.async_copy(rel_hbm.at[ir], buf_r.at[par, r],
                                 sems.at[par], priority=pr)
                pltpu.async_copy(ent_hbm.at[it], buf_t.at[par, r],
                                 sems.at[par], priority=pt)
            return carry

        lax.fori_loop(0, BLK // 2, enqueue, 0, unroll=4)

    par = lax.rem(k, 2)
    nxt = lax.rem(k + 1, 2)

    @pl.when(k == 0)
    def _():
        issue_block(0, 0)

    @pl.when(k + 1 < NBLK)
    def _():
        issue_block(k + 1, nxt)

    for buf in (buf_h, buf_r, buf_t):
        pltpu.make_async_copy(
            ent_hbm.at[pl.ds(0, BLK)], buf.at[par], sems.at[par]).wait()

    h = buf_h[par]
    r = buf_r[par]
    t = buf_t[par]
    d = jnp.abs(h + r - t)
    out_ref[...] = GAMMA - jnp.sum(d, axis=1, keepdims=True)


@jax.jit
def _score(heads, rels, tails, entity_embedding, relation_embedding):
    grid_spec = pltpu.PrefetchScalarGridSpec(
        num_scalar_prefetch=3,
        grid=(NBLK,),
        in_specs=[
            pl.BlockSpec(memory_space=pl.ANY),
            pl.BlockSpec(memory_space=pl.ANY),
        ],
        out_specs=pl.BlockSpec((BLK, 1), lambda k, *p: (k, 0)),
        scratch_shapes=[
            pltpu.VMEM((2, BLK, HIDDEN), jnp.float32),
            pltpu.VMEM((2, BLK, HIDDEN), jnp.float32),
            pltpu.VMEM((2, BLK, HIDDEN), jnp.float32),
            pltpu.SemaphoreType.DMA((2,)),
        ],
    )
    fn = pl.pallas_call(
        _body,
        grid_spec=grid_spec,
        out_shape=jax.ShapeDtypeStruct((BATCH, 1), jnp.float32),
        compiler_params=pltpu.CompilerParams(
            dimension_semantics=("arbitrary",)),
    )
    return fn(heads, rels, tails, entity_embedding, relation_embedding)


def kernel(sample, entity_embedding, relation_embedding):
    sample = sample.astype(jnp.int32)
    heads = sample[:, 0]
    rels = sample[:, 1]
    tails = sample[:, 2]
    return _score(heads, rels, tails, entity_embedding, relation_embedding)
